# Initial kernel scaffold; baseline (speedup 1.0000x reference)
import jax
import jax.numpy as jnp
from jax.experimental import pallas as pl


def kernel(node_embedding, triplets, labels, relational_embedding):
    raise NotImplementedError("write your pallas kernel here")


# speedup vs baseline: 2.6975x; 2.6975x over previous
import functools

import jax
import jax.numpy as jnp
from jax import lax
from jax.experimental import pallas as pl
from jax.experimental.pallas import tpu as pltpu
from jax.experimental.pallas import tpu_sc as plsc

NUM_NODES = 100000
NUM_RELS = 100000
D = 64
N_TRIPLETS = 1000000
REG = 0.01

LANES = 16
N_PAD = 1 << 20
IDX_COLS = 128
IDX_ROWS = N_PAD // IDX_COLS
NC, NS = 2, 16
NW = NC * NS
ROWS_PER_TILE = IDX_ROWS // NW
SUP = 16
N_SUP = ROWS_PER_TILE // SUP


def _sc_scores_body(node_hbm, rel_hbm, h_hbm, r_hbm, t_hbm, out_hbm,
                    hi_v, ri_v, ti_v, hbuf, rbuf, tbuf, spart, sv,
                    sem0, sem1):
    wid = lax.axis_index("s") * NC + lax.axis_index("c")
    base = wid * ROWS_PER_TILE
    sems = (sem0, sem1)

    def fire(c, hi, ri, ti):
        slot = c % 2
        sem = sems[slot]
        ch = pltpu.async_copy(node_hbm.at[hi.at[c]], hbuf.at[slot], sem)
        cr = pltpu.async_copy(rel_hbm.at[ri.at[c]], rbuf.at[slot], sem)
        ct = pltpu.async_copy(node_hbm.at[ti.at[c]], tbuf.at[slot], sem)
        return (ch, cr, ct)

    def compute(c):
        slot = c % 2
        hb = hbuf.at[slot]
        rb = rbuf.at[slot]
        tb = tbuf.at[slot]

        def row_body(i, _):
            acc = (hb[i, pl.ds(0, LANES)] * rb[i, pl.ds(0, LANES)]
                   * tb[i, pl.ds(0, LANES)])
            for sgm in range(1, D // LANES):
                o = sgm * LANES
                acc = acc + (hb[i, pl.ds(o, LANES)] * rb[i, pl.ds(o, LANES)]
                             * tb[i, pl.ds(o, LANES)])
            spart[pl.ds(i * 17, LANES)] = acc
            return 0

        lax.fori_loop(0, IDX_COLS, row_body, 0, unroll=4)

        iota = lax.iota(jnp.int32, LANES)

        def grp_body(g, _):
            flat0 = (g * LANES + iota) * 17
            acc = plsc.load_gather(spart, [flat0])
            for k in range(1, LANES):
                acc = acc + plsc.load_gather(spart, [flat0 + k])
            sv[c, pl.ds(g * LANES, LANES)] = acc
            return 0

        lax.fori_loop(0, IDX_COLS // LANES, grp_body, 0)

    def super_body(s, _):
        row0 = base + s * SUP
        pltpu.sync_copy(h_hbm.at[pl.ds(row0, SUP)], hi_v)
        pltpu.sync_copy(r_hbm.at[pl.ds(row0, SUP)], ri_v)
        pltpu.sync_copy(t_hbm.at[pl.ds(row0, SUP)], ti_v)
        pending = fire(0, hi_v, ri_v, ti_v)
        for c in range(SUP):
            if c + 1 < SUP:
                nxt = fire(c + 1, hi_v, ri_v, ti_v)
            for cp in pending:
                cp.wait()
            compute(c)
            if c + 1 < SUP:
                pending = nxt
        pltpu.sync_copy(sv, out_hbm.at[pl.ds(row0, SUP)])
        return 0

    lax.fori_loop(0, N_SUP, super_body, 0)


def _sc_scores(node_emb, rel_emb, h2d, r2d, t2d):
    mesh = plsc.VectorSubcoreMesh(core_axis_name="c", subcore_axis_name="s")
    fn = pl.kernel(
        _sc_scores_body,
        out_type=jax.ShapeDtypeStruct((IDX_ROWS, IDX_COLS), jnp.float32),
        mesh=mesh,
        compiler_params=pltpu.CompilerParams(
            needs_layout_passes=False, use_tc_tiling_on_sc=False),
        scratch_types=[
            pltpu.VMEM((SUP, IDX_COLS), jnp.int32),
            pltpu.VMEM((SUP, IDX_COLS), jnp.int32),
            pltpu.VMEM((SUP, IDX_COLS), jnp.int32),
            pltpu.VMEM((2, IDX_COLS, D), jnp.float32),
            pltpu.VMEM((2, IDX_COLS, D), jnp.float32),
            pltpu.VMEM((2, IDX_COLS, D), jnp.float32),
            pltpu.VMEM((IDX_COLS * 17,), jnp.float32),
            pltpu.VMEM((SUP, IDX_COLS), jnp.float32),
            pltpu.SemaphoreType.DMA,
            pltpu.SemaphoreType.DMA,
        ],
    )
    return fn(node_emb, rel_emb, h2d, r2d, t2d)


_G = 8
_SC_BLK = IDX_ROWS // _G


def _ce_body(sb, lb, out_ref, acc_ref):
    step = pl.program_id(0)

    @pl.when(step == 0)
    def _init():
        acc_ref[0] = 0.0

    s = sb[...]
    y = lb[...]
    rows = lax.broadcasted_iota(jnp.int32, (_SC_BLK, IDX_COLS), 0) + step * _SC_BLK
    idx = rows * IDX_COLS + lax.broadcasted_iota(jnp.int32, (_SC_BLK, IDX_COLS), 1)
    valid = idx < N_TRIPLETS
    ce = jnp.maximum(s, 0.0) - s * y + jnp.log1p(jnp.exp(-jnp.abs(s)))
    ce = jnp.where(valid, ce, 0.0)
    acc_ref[0] = acc_ref[0] + jnp.sum(ce)

    @pl.when(step == _G - 1)
    def _fin():
        out_ref[0, 0] = acc_ref[0] / N_TRIPLETS


def _tc_ce(scores2d, labels2d):
    return pl.pallas_call(
        _ce_body,
        grid=(_G,),
        in_specs=[
            pl.BlockSpec((_SC_BLK, IDX_COLS), lambda i: (i, 0)),
            pl.BlockSpec((_SC_BLK, IDX_COLS), lambda i: (i, 0)),
        ],
        out_specs=pl.BlockSpec(memory_space=pltpu.SMEM),
        out_shape=jax.ShapeDtypeStruct((1, 1), jnp.float32),
        scratch_shapes=[pltpu.SMEM((1,), jnp.float32)],
    )(scores2d, labels2d)


_RG = 25
_REG_BLK = NUM_NODES // _RG


def _reg_body(nb, rb, out_ref, acc_ref):
    step = pl.program_id(0)

    @pl.when(step == 0)
    def _init():
        acc_ref[0] = 0.0
        acc_ref[1] = 0.0

    acc_ref[0] = acc_ref[0] + jnp.sum(nb[...] * nb[...])
    acc_ref[1] = acc_ref[1] + jnp.sum(rb[...] * rb[...])

    @pl.when(step == _RG - 1)
    def _fin():
        out_ref[0, 0] = REG * (acc_ref[0] / (NUM_NODES * D)
                               + acc_ref[1] / (NUM_RELS * D))


def _tc_reg(node_emb, rel_emb):
    return pl.pallas_call(
        _reg_body,
        grid=(_RG,),
        in_specs=[
            pl.BlockSpec((_REG_BLK, D), lambda i: (i, 0)),
            pl.BlockSpec((_REG_BLK, D), lambda i: (i, 0)),
        ],
        out_specs=pl.BlockSpec(memory_space=pltpu.SMEM),
        out_shape=jax.ShapeDtypeStruct((1, 1), jnp.float32),
        scratch_shapes=[pltpu.SMEM((2,), jnp.float32)],
    )(node_emb, rel_emb)


def kernel(node_embedding, triplets, labels, relational_embedding):
    tri = triplets.astype(jnp.int32)
    pad = N_PAD - N_TRIPLETS
    h2d = jnp.pad(tri[:, 0], (0, pad)).reshape(IDX_ROWS, IDX_COLS)
    r2d = jnp.pad(tri[:, 1], (0, pad)).reshape(IDX_ROWS, IDX_COLS)
    t2d = jnp.pad(tri[:, 2], (0, pad)).reshape(IDX_ROWS, IDX_COLS)
    lab2d = jnp.pad(labels.astype(jnp.float32), (0, pad)).reshape(IDX_ROWS, IDX_COLS)
    scores2d = _sc_scores(node_embedding, relational_embedding, h2d, r2d, t2d)
    ce = _tc_ce(scores2d, lab2d)
    reg = _tc_reg(node_embedding, relational_embedding)
    return ce[0, 0] + reg[0, 0]


# speedup vs baseline: 2.7013x; 1.0014x over previous
import functools

import jax
import jax.numpy as jnp
from jax import lax
from jax.experimental import pallas as pl
from jax.experimental.pallas import tpu as pltpu
from jax.experimental.pallas import tpu_sc as plsc

NUM_NODES = 100000
NUM_RELS = 100000
D = 64
N_TRIPLETS = 1000000
REG = 0.01

LANES = 16
N_PAD = 1 << 20
IDX_COLS = 128
IDX_ROWS = N_PAD // IDX_COLS
NC, NS = 2, 16
NW = NC * NS
ROWS_PER_TILE = IDX_ROWS // NW
SUP = 16
N_SUP = ROWS_PER_TILE // SUP


def _sc_scores_body(node_hbm, rel_hbm, h_hbm, r_hbm, t_hbm, out_hbm,
                    hi_v, ri_v, ti_v, hbuf, rbuf, tbuf, spart, sv,
                    sem0, sem1):
    wid = lax.axis_index("s") * NC + lax.axis_index("c")
    base = wid * ROWS_PER_TILE
    sems = (sem0, sem1)

    def fire(c, hi, ri, ti):
        slot = c % 2
        sem = sems[slot]
        ch = pltpu.async_copy(node_hbm.at[hi.at[c]], hbuf.at[slot], sem)
        cr = pltpu.async_copy(rel_hbm.at[ri.at[c]], rbuf.at[slot], sem)
        ct = pltpu.async_copy(node_hbm.at[ti.at[c]], tbuf.at[slot], sem)
        return (ch, cr, ct)

    def compute(c):
        slot = c % 2
        hb = hbuf.at[slot]
        rb = rbuf.at[slot]
        tb = tbuf.at[slot]

        def row_body(i):
            acc = (hb[i, pl.ds(0, LANES)] * rb[i, pl.ds(0, LANES)]
                   * tb[i, pl.ds(0, LANES)])
            for sgm in range(1, D // LANES):
                o = sgm * LANES
                acc = acc + (hb[i, pl.ds(o, LANES)] * rb[i, pl.ds(o, LANES)]
                             * tb[i, pl.ds(o, LANES)])
            spart[pl.ds(i * 17, LANES)] = acc

        plsc.parallel_loop(0, IDX_COLS, unroll=4)(row_body)

        iota = lax.iota(jnp.int32, LANES)

        def grp_body(g):
            flat0 = (g * LANES + iota) * 17
            acc = plsc.load_gather(spart, [flat0])
            for k in range(1, LANES):
                acc = acc + plsc.load_gather(spart, [flat0 + k])
            sv[c, pl.ds(g * LANES, LANES)] = acc

        plsc.parallel_loop(0, IDX_COLS // LANES, unroll=2)(grp_body)

    def super_body(s, _):
        row0 = base + s * SUP
        pltpu.sync_copy(h_hbm.at[pl.ds(row0, SUP)], hi_v)
        pltpu.sync_copy(r_hbm.at[pl.ds(row0, SUP)], ri_v)
        pltpu.sync_copy(t_hbm.at[pl.ds(row0, SUP)], ti_v)
        pending = fire(0, hi_v, ri_v, ti_v)
        for c in range(SUP):
            if c + 1 < SUP:
                nxt = fire(c + 1, hi_v, ri_v, ti_v)
            for cp in pending:
                cp.wait()
            compute(c)
            if c + 1 < SUP:
                pending = nxt
        pltpu.sync_copy(sv, out_hbm.at[pl.ds(row0, SUP)])
        return 0

    lax.fori_loop(0, N_SUP, super_body, 0)


def _sc_scores(node_emb, rel_emb, h2d, r2d, t2d):
    mesh = plsc.VectorSubcoreMesh(core_axis_name="c", subcore_axis_name="s")
    fn = pl.kernel(
        _sc_scores_body,
        out_type=jax.ShapeDtypeStruct((IDX_ROWS, IDX_COLS), jnp.float32),
        mesh=mesh,
        compiler_params=pltpu.CompilerParams(
            needs_layout_passes=False, use_tc_tiling_on_sc=False),
        scratch_types=[
            pltpu.VMEM((SUP, IDX_COLS), jnp.int32),
            pltpu.VMEM((SUP, IDX_COLS), jnp.int32),
            pltpu.VMEM((SUP, IDX_COLS), jnp.int32),
            pltpu.VMEM((2, IDX_COLS, D), jnp.float32),
            pltpu.VMEM((2, IDX_COLS, D), jnp.float32),
            pltpu.VMEM((2, IDX_COLS, D), jnp.float32),
            pltpu.VMEM((IDX_COLS * 17,), jnp.float32),
            pltpu.VMEM((SUP, IDX_COLS), jnp.float32),
            pltpu.SemaphoreType.DMA,
            pltpu.SemaphoreType.DMA,
        ],
    )
    return fn(node_emb, rel_emb, h2d, r2d, t2d)


_G = 8
_SC_BLK = IDX_ROWS // _G


def _ce_body(sb, lb, out_ref, acc_ref):
    step = pl.program_id(0)

    @pl.when(step == 0)
    def _init():
        acc_ref[0] = 0.0

    s = sb[...]
    y = lb[...]
    rows = lax.broadcasted_iota(jnp.int32, (_SC_BLK, IDX_COLS), 0) + step * _SC_BLK
    idx = rows * IDX_COLS + lax.broadcasted_iota(jnp.int32, (_SC_BLK, IDX_COLS), 1)
    valid = idx < N_TRIPLETS
    ce = jnp.maximum(s, 0.0) - s * y + jnp.log1p(jnp.exp(-jnp.abs(s)))
    ce = jnp.where(valid, ce, 0.0)
    acc_ref[0] = acc_ref[0] + jnp.sum(ce)

    @pl.when(step == _G - 1)
    def _fin():
        out_ref[0, 0] = acc_ref[0] / N_TRIPLETS


def _tc_ce(scores2d, labels2d):
    return pl.pallas_call(
        _ce_body,
        grid=(_G,),
        in_specs=[
            pl.BlockSpec((_SC_BLK, IDX_COLS), lambda i: (i, 0)),
            pl.BlockSpec((_SC_BLK, IDX_COLS), lambda i: (i, 0)),
        ],
        out_specs=pl.BlockSpec(memory_space=pltpu.SMEM),
        out_shape=jax.ShapeDtypeStruct((1, 1), jnp.float32),
        scratch_shapes=[pltpu.SMEM((1,), jnp.float32)],
    )(scores2d, labels2d)


_RG = 25
_REG_BLK = NUM_NODES // _RG


def _reg_body(nb, rb, out_ref, acc_ref):
    step = pl.program_id(0)

    @pl.when(step == 0)
    def _init():
        acc_ref[0] = 0.0
        acc_ref[1] = 0.0

    acc_ref[0] = acc_ref[0] + jnp.sum(nb[...] * nb[...])
    acc_ref[1] = acc_ref[1] + jnp.sum(rb[...] * rb[...])

    @pl.when(step == _RG - 1)
    def _fin():
        out_ref[0, 0] = REG * (acc_ref[0] / (NUM_NODES * D)
                               + acc_ref[1] / (NUM_RELS * D))


def _tc_reg(node_emb, rel_emb):
    return pl.pallas_call(
        _reg_body,
        grid=(_RG,),
        in_specs=[
            pl.BlockSpec((_REG_BLK, D), lambda i: (i, 0)),
            pl.BlockSpec((_REG_BLK, D), lambda i: (i, 0)),
        ],
        out_specs=pl.BlockSpec(memory_space=pltpu.SMEM),
        out_shape=jax.ShapeDtypeStruct((1, 1), jnp.float32),
        scratch_shapes=[pltpu.SMEM((2,), jnp.float32)],
    )(node_emb, rel_emb)


def kernel(node_embedding, triplets, labels, relational_embedding):
    tri = triplets.astype(jnp.int32)
    pad = N_PAD - N_TRIPLETS
    h2d = jnp.pad(tri[:, 0], (0, pad)).reshape(IDX_ROWS, IDX_COLS)
    r2d = jnp.pad(tri[:, 1], (0, pad)).reshape(IDX_ROWS, IDX_COLS)
    t2d = jnp.pad(tri[:, 2], (0, pad)).reshape(IDX_ROWS, IDX_COLS)
    lab2d = jnp.pad(labels.astype(jnp.float32), (0, pad)).reshape(IDX_ROWS, IDX_COLS)
    scores2d = _sc_scores(node_embedding, relational_embedding, h2d, r2d, t2d)
    ce = _tc_ce(scores2d, lab2d)
    reg = _tc_reg(node_embedding, relational_embedding)
    return ce[0, 0] + reg[0, 0]
